# Initial kernel scaffold; baseline (speedup 1.0000x reference)
import jax
import jax.numpy as jnp
from jax.experimental import pallas as pl


def kernel(x, edge_index, edge_weight, W1, b1, gamma, beta, Wc, bc):
    raise NotImplementedError("write your pallas kernel here")


# speedup vs baseline: 8.4812x; 8.4812x over previous
import functools

import jax
import jax.numpy as jnp
from jax import lax
from jax.experimental import pallas as pl
from jax.experimental.pallas import tpu as pltpu
from jax.experimental.pallas import tpu_sc as plsc

N = 10000
D = 128
D_OUT = 64
E = 320000
EPS = 1e-5

NC = 2
NS = 16
NW = NC * NS
EPW = E // NW
CHUNK = 80
NCHUNK = EPW // CHUNK
NP = 10240
RPT = NP // NS


def _spmm_sc(feat, row3, col3, w3, zeros):
    mesh = plsc.VectorSubcoreMesh(core_axis_name="c", subcore_axis_name="s")

    @functools.partial(
        pl.kernel,
        out_type=jax.ShapeDtypeStruct((NC, N, D), jnp.float32),
        mesh=mesh,
        scratch_types=[
            pltpu.VMEM_SHARED((N, D), jnp.float32),
            pltpu.VMEM((NCHUNK, CHUNK), jnp.int32),
            pltpu.VMEM((EPW,), jnp.int32),
            pltpu.VMEM((CHUNK, D), jnp.float32),
            pltpu.VMEM((CHUNK, D), jnp.float32),
            pltpu.VMEM((CHUNK * 16,), jnp.float32),
            pltpu.VMEM((CHUNK * 16,), jnp.float32),
            pltpu.SemaphoreType.DMA,
            pltpu.SemaphoreType.DMA,
            pltpu.SemaphoreType.DMA,
            pltpu.SemaphoreType.DMA,
        ],
    )
    def k(feat_hbm, row_hbm, col_hbm, w_hbm, zero_hbm, out_hbm,
          acc, rowv, colv, rows0, rows1, wb0, wb1, sg0, sg1, sw0, sw1):
        cid = lax.axis_index("c")
        sid = lax.axis_index("s")
        wid = cid * NS + sid
        z0 = sid * 624

        @pl.when(sid < NS - 1)
        def _():
            pltpu.sync_copy(zero_hbm.at[pl.ds(z0, 624)],
                            acc.at[pl.ds(z0, 624)])

        @pl.when(sid == NS - 1)
        def _():
            pltpu.sync_copy(zero_hbm.at[pl.ds(9360, 640)],
                            acc.at[pl.ds(9360, 640)])

        pltpu.sync_copy(row_hbm.at[wid], rowv)
        pltpu.sync_copy(col_hbm.at[wid], colv)
        plsc.subcore_barrier()

        rbufs = (rows0, rows1)
        wbufs = (wb0, wb1)
        gsems = (sg0, sg1)
        wsems = (sw0, sw1)

        def issue_gather(kc, b):
            idx = colv.at[pl.ds(kc * CHUNK, CHUNK)]
            pltpu.async_copy(feat_hbm.at[idx], rbufs[b], gsems[b])
            pltpu.async_copy(w_hbm.at[wid, kc], wbufs[b], wsems[b])

        def drain_gather(b):
            pltpu.make_async_copy(
                feat_hbm.at[pl.ds(0, CHUNK)], rbufs[b], gsems[b]).wait()
            pltpu.make_async_copy(w_hbm.at[0, 0], wbufs[b], wsems[b]).wait()

        def compute_scatter(kc, b):
            rows, wb = rbufs[b], wbufs[b]

            def edge_body(i, c):
                wv = wb[pl.ds(i * 16, 16)]
                for j in range(D // 16):
                    rows[i, pl.ds(j * 16, 16)] = rows[i, pl.ds(j * 16, 16)] * wv
                return c

            lax.fori_loop(0, CHUNK, edge_body, 0)
            pltpu.sync_copy(rows, acc.at[rowv.at[kc]], add=True)

        issue_gather(0, 0)
        issue_gather(1, 1)

        def pair(kp, carry):
            kc0 = 2 * kp
            drain_gather(0)
            compute_scatter(kc0, 0)
            issue_gather(kc0 + 2, 0)
            drain_gather(1)
            compute_scatter(kc0 + 1, 1)

            @pl.when(kp < (NCHUNK - 1) // 2 - 1)
            def _():
                issue_gather(kc0 + 3, 1)

            return carry

        lax.fori_loop(0, (NCHUNK - 1) // 2, pair, 0)

        drain_gather(0)
        compute_scatter(NCHUNK - 1, 0)

        plsc.subcore_barrier()

        @pl.when(sid < NS - 1)
        def _():
            pltpu.sync_copy(acc.at[pl.ds(z0, 624)],
                            out_hbm.at[cid, pl.ds(z0, 624)])

        @pl.when(sid == NS - 1)
        def _():
            pltpu.sync_copy(acc.at[pl.ds(9360, 640)],
                            out_hbm.at[cid, pl.ds(9360, 640)])

    return k(feat, row3, col3, w3, zeros)


def _fuse1(z1, w1, b1, gamma, beta):
    def body(zr, w1r, b1r, gr, br, out):
        z = zr[0] + zr[1]
        z = lax.dot_general(z, w1r[...], (((1,), (1,)), ((), ())),
                            preferred_element_type=jnp.float32)
        z = z + b1r[...]
        mean = jnp.mean(z, axis=0, keepdims=True)
        zc = z - mean
        var = jnp.mean(zc * zc, axis=0, keepdims=True)
        h = zc * lax.rsqrt(var + EPS) * gr[...] + br[...]
        out[...] = jnp.maximum(h, 0.0)

    return pl.pallas_call(
        body,
        out_shape=jax.ShapeDtypeStruct((N, D), jnp.float32),
    )(z1, w1, b1.reshape(1, D), gamma.reshape(1, D), beta.reshape(1, D))


def _fuse2(z2, wc, bc):
    def body(zr, wcr, bcr, out):
        z = zr[0] + zr[1]
        z = lax.dot_general(z, wcr[...], (((1,), (1,)), ((), ())),
                            preferred_element_type=jnp.float32)
        z = z + bcr[...]
        m = jnp.max(z, axis=1, keepdims=True)
        zs = z - m
        lse = jnp.log(jnp.sum(jnp.exp(zs), axis=1, keepdims=True))
        out[...] = zs - lse

    return pl.pallas_call(
        body,
        out_shape=jax.ShapeDtypeStruct((N, D_OUT), jnp.float32),
    )(z2, wc, bc.reshape(1, D_OUT))


def kernel(x, edge_index, edge_weight, W1, b1, gamma, beta, Wc, bc):
    row3 = edge_index[0].astype(jnp.int32).reshape(NW, NCHUNK, CHUNK)
    col3 = edge_index[1].astype(jnp.int32).reshape(NW, EPW)
    w3 = jnp.broadcast_to(edge_weight[:, None], (E, 16)).reshape(
        NW, NCHUNK, CHUNK * 16)
    zeros = jnp.zeros((N, D), jnp.float32)
    z1 = _spmm_sc(x, row3, col3, w3, zeros)
    h = _fuse1(z1, W1, b1, gamma, beta)
    z2 = _spmm_sc(h, row3, col3, w3, zeros)
    return _fuse2(z2, Wc, bc)


# speedup vs baseline: 10.2852x; 1.2127x over previous
import functools

import jax
import jax.numpy as jnp
from jax import lax
from jax.experimental import pallas as pl
from jax.experimental.pallas import tpu as pltpu
from jax.experimental.pallas import tpu_sc as plsc

N = 10000
D = 128
D_OUT = 64
E = 320000
EPS = 1e-5

NC = 2
NS = 16
NW = NC * NS
EPW = E // NW
CHUNK = 40
NCHUNK = EPW // CHUNK
NP = 10240
RPT = NP // NS


def _spmm_sc(feat, row2, col2, w2, zeros):
    mesh = plsc.VectorSubcoreMesh(core_axis_name="c", subcore_axis_name="s")
    NBUF = 4

    @functools.partial(
        pl.kernel,
        out_type=jax.ShapeDtypeStruct((NC, N, D), jnp.float32),
        mesh=mesh,
        scratch_types=[
            pltpu.VMEM_SHARED((N, D), jnp.float32),
            pltpu.VMEM((EPW,), jnp.int32),
            pltpu.VMEM((EPW,), jnp.int32),
            pltpu.VMEM((CHUNK, D), jnp.float32),
            pltpu.VMEM((CHUNK, D), jnp.float32),
            pltpu.VMEM((CHUNK, D), jnp.float32),
            pltpu.VMEM((CHUNK, D), jnp.float32),
            pltpu.VMEM((CHUNK * 16,), jnp.float32),
            pltpu.VMEM((CHUNK * 16,), jnp.float32),
            pltpu.VMEM((CHUNK * 16,), jnp.float32),
            pltpu.VMEM((CHUNK * 16,), jnp.float32),
            pltpu.SemaphoreType.DMA,
            pltpu.SemaphoreType.DMA,
            pltpu.SemaphoreType.DMA,
            pltpu.SemaphoreType.DMA,
            pltpu.SemaphoreType.DMA,
            pltpu.SemaphoreType.DMA,
            pltpu.SemaphoreType.DMA,
            pltpu.SemaphoreType.DMA,
        ],
    )
    def k(feat_hbm, row_hbm, col_hbm, w_hbm, zero_hbm, out_hbm,
          acc, rowv, colv, r0b, r1b, r2b, r3b, wb0, wb1, wb2, wb3,
          sg0, sg1, sg2, sg3, ss0, ss1, ss2, ss3):
        cid = lax.axis_index("c")
        sid = lax.axis_index("s")
        wid = cid * NS + sid
        z0 = sid * 624

        @pl.when(sid < NS - 1)
        def _():
            pltpu.sync_copy(zero_hbm.at[pl.ds(z0, 624)],
                            acc.at[pl.ds(z0, 624)])

        @pl.when(sid == NS - 1)
        def _():
            pltpu.sync_copy(zero_hbm.at[pl.ds(9360, 640)],
                            acc.at[pl.ds(9360, 640)])

        pltpu.sync_copy(row_hbm.at[wid], rowv)
        pltpu.sync_copy(col_hbm.at[wid], colv)
        plsc.subcore_barrier()

        bufs = (r0b, r1b, r2b, r3b)
        wbufs = (wb0, wb1, wb2, wb3)
        gsems = (sg0, sg1, sg2, sg3)
        ssems = (ss0, ss1, ss2, ss3)

        def issue_gather(kc, b):
            idx = colv.at[pl.ds(kc * CHUNK, CHUNK)]
            pltpu.async_copy(feat_hbm.at[idx], bufs[b], gsems[b])
            pltpu.async_copy(w_hbm.at[wid, kc], wbufs[b], gsems[b])

        def drain_gather(b):
            pltpu.make_async_copy(
                feat_hbm.at[pl.ds(0, CHUNK)], bufs[b], gsems[b]).wait()
            pltpu.make_async_copy(
                w_hbm.at[0, 0], wbufs[b], gsems[b]).wait()

        def issue_scatter(kc, b):
            idx = rowv.at[pl.ds(kc * CHUNK, CHUNK)]
            pltpu.async_copy(bufs[b], acc.at[idx], ssems[b], add=True)

        def drain_scatter(b):
            pltpu.make_async_copy(
                feat_hbm.at[pl.ds(0, CHUNK)], bufs[b], ssems[b]).wait()

        def compute(kc, b):
            rows = bufs[b]

            wb = wbufs[b]

            def edge_body(i, c):
                s = wb[pl.ds(i * 16, 16)]
                for j in range(D // 16):
                    rows[i, pl.ds(j * 16, 16)] = rows[i, pl.ds(j * 16, 16)] * s
                return c

            lax.fori_loop(0, CHUNK, edge_body, 0)

        issue_gather(0, 0)
        issue_gather(1, 1)
        issue_gather(2, 2)

        def step(kc, kq, j):
            b = j
            bp = (j + 3) % NBUF
            drain_gather(b)
            compute(kc, b)
            issue_scatter(kc, b)
            if j == 0:
                @pl.when(kq > 0)
                def _():
                    drain_scatter(bp)

                issue_gather(kc + 3, bp)
            else:
                drain_scatter(bp)

                @pl.when(kc + 3 < NCHUNK)
                def _():
                    issue_gather(kc + 3, bp)

        def quad(kq, carry):
            for j in range(NBUF):
                step(kq * NBUF + j, kq, j)
            return carry

        lax.fori_loop(0, NCHUNK // NBUF, quad, 0)

        for t in range(NCHUNK - (NCHUNK // NBUF) * NBUF):
            kc = (NCHUNK // NBUF) * NBUF + t
            b = kc % NBUF
            drain_gather(b)
            compute(kc, b)
            issue_scatter(kc, b)
            drain_scatter((b + 3) % NBUF)
        drain_scatter((NCHUNK - 1) % NBUF)

        plsc.subcore_barrier()

        @pl.when(sid < NS - 1)
        def _():
            pltpu.sync_copy(acc.at[pl.ds(z0, 624)],
                            out_hbm.at[cid, pl.ds(z0, 624)])

        @pl.when(sid == NS - 1)
        def _():
            pltpu.sync_copy(acc.at[pl.ds(9360, 640)],
                            out_hbm.at[cid, pl.ds(9360, 640)])

    return k(feat, row2, col2, w2, zeros)


def _fuse1(z1, w1, b1, gamma, beta):
    def body(zr, w1r, b1r, gr, br, out):
        z = zr[0] + zr[1]
        z = lax.dot_general(z, w1r[...], (((1,), (1,)), ((), ())),
                            preferred_element_type=jnp.float32)
        z = z + b1r[...]
        mean = jnp.mean(z, axis=0, keepdims=True)
        zc = z - mean
        var = jnp.mean(zc * zc, axis=0, keepdims=True)
        h = zc * lax.rsqrt(var + EPS) * gr[...] + br[...]
        out[...] = jnp.maximum(h, 0.0)

    return pl.pallas_call(
        body,
        out_shape=jax.ShapeDtypeStruct((N, D), jnp.float32),
    )(z1, w1, b1.reshape(1, D), gamma.reshape(1, D), beta.reshape(1, D))


def _fuse2(z2, wc, bc):
    def body(zr, wcr, bcr, out):
        z = zr[0] + zr[1]
        z = lax.dot_general(z, wcr[...], (((1,), (1,)), ((), ())),
                            preferred_element_type=jnp.float32)
        z = z + bcr[...]
        m = jnp.max(z, axis=1, keepdims=True)
        zs = z - m
        lse = jnp.log(jnp.sum(jnp.exp(zs), axis=1, keepdims=True))
        out[...] = zs - lse

    return pl.pallas_call(
        body,
        out_shape=jax.ShapeDtypeStruct((N, D_OUT), jnp.float32),
    )(z2, wc, bc.reshape(1, D_OUT))


def kernel(x, edge_index, edge_weight, W1, b1, gamma, beta, Wc, bc):
    row2 = edge_index[0].astype(jnp.int32).reshape(NW, EPW)
    col2 = edge_index[1].astype(jnp.int32).reshape(NW, EPW)
    w2 = jnp.broadcast_to(edge_weight[:, None], (E, 16)).reshape(
        NW, NCHUNK, CHUNK * 16)
    zeros = jnp.zeros((N, D), jnp.float32)
    z1 = _spmm_sc(x, row2, col2, w2, zeros)
    h = _fuse1(z1, W1, b1, gamma, beta)
    z2 = _spmm_sc(h, row2, col2, w2, zeros)
    return _fuse2(z2, Wc, bc)
